# baseline (speedup 1.0000x reference)
import functools

import jax
import jax.numpy as jnp
from jax import lax
from jax.experimental import pallas as pl
from jax.experimental.pallas import tpu as pltpu
from jax.experimental.pallas import tpu_sc as plsc

N = 10000
E = 320000
NCL = 3

NC = 2
NS = 16
NW = NC * NS
CH = 128
NCH = 80
EPW = NCH * CH
HCH = NCH // 2
NP = 10240
RPT = NP // NS
ZR = 128

_mesh = plsc.VectorSubcoreMesh(core_axis_name="c", subcore_axis_name="s")


def _sc_worker_ids():
    cid = lax.axis_index("c")
    sid = lax.axis_index("s")
    wid = sid * NC + cid
    return cid, sid, wid


@functools.partial(
    pl.kernel,
    mesh=_mesh,
    out_type=jax.ShapeDtypeStruct((2, NC, NP, 128), jnp.float32),
    scratch_types=[
        pltpu.VMEM_SHARED((NP, 128), jnp.float32),
        pltpu.VMEM((NCH, CH), jnp.int32),
        pltpu.VMEM((CH, 128), jnp.float32),
    ],
)
def _deg_kernel(dst1, dst2, ones_h, zeros_h, out, acc, dall, ones_v):
    cid, sid, wid = _sc_worker_ids()
    rbase = sid * RPT
    pltpu.sync_copy(ones_h, ones_v)
    for v, dref in ((0, dst1), (1, dst2)):
        for j in range(RPT // ZR):
            pltpu.sync_copy(zeros_h, acc.at[pl.ds(rbase + j * ZR, ZR)])
        pltpu.sync_copy(dref.at[wid], dall)
        plsc.subcore_barrier()

        def chunk(i, carry):
            pltpu.sync_copy(ones_v, acc.at[dall.at[i]], add=True)
            return carry

        lax.fori_loop(0, NCH, chunk, 0)
        plsc.subcore_barrier()
        pltpu.sync_copy(acc.at[pl.ds(rbase, RPT)],
                        out.at[v, cid, pl.ds(rbase, RPT)])
        plsc.subcore_barrier()


def _make_seg_kernel(n_pass, view_of_pass):
    def body(*refs):
        ys = refs[:n_pass]
        srcs = refs[n_pass:n_pass + 2]
        dsts = refs[n_pass + 2:n_pass + 4]
        zeros_h = refs[n_pass + 4]
        out = refs[n_pass + 5]
        acc, sall, dall, rows0, rows1, sem0, sem1 = refs[n_pass + 6:]
        cid, sid, wid = _sc_worker_ids()
        rbase = sid * RPT
        for p in range(n_pass):
            v = view_of_pass[p]
            yref = ys[p]
            for j in range(RPT // ZR):
                pltpu.sync_copy(zeros_h, acc.at[pl.ds(rbase + j * ZR, ZR)])
            plsc.subcore_barrier()

            for half in range(2):
                cbase = half * HCH
                pltpu.sync_copy(srcs[v].at[wid, pl.ds(cbase, HCH)], sall)
                pltpu.sync_copy(dsts[v].at[wid, pl.ds(cbase, HCH)], dall)

                pltpu.async_copy(yref.at[sall.at[0]], rows0, sem0)

                def pair(t, carry, yref=yref):
                    c0 = 2 * t
                    pltpu.async_copy(yref.at[sall.at[c0 + 1]], rows1, sem1)
                    pltpu.make_async_copy(yref.at[sall.at[c0]], rows0,
                                          sem0).wait()
                    pltpu.sync_copy(rows0, acc.at[dall.at[c0]], add=True)
                    pltpu.async_copy(yref.at[sall.at[c0 + 2]], rows0, sem0)
                    pltpu.make_async_copy(yref.at[sall.at[c0 + 1]], rows1,
                                          sem1).wait()
                    pltpu.sync_copy(rows1, acc.at[dall.at[c0 + 1]], add=True)
                    return carry

                lax.fori_loop(0, HCH // 2 - 1, pair, 0)
                cl = HCH - 2
                pltpu.async_copy(yref.at[sall.at[cl + 1]], rows1, sem1)
                pltpu.make_async_copy(yref.at[sall.at[cl]], rows0, sem0).wait()
                pltpu.sync_copy(rows0, acc.at[dall.at[cl]], add=True)
                pltpu.make_async_copy(yref.at[sall.at[cl + 1]], rows1,
                                      sem1).wait()
                pltpu.sync_copy(rows1, acc.at[dall.at[cl + 1]], add=True)

            plsc.subcore_barrier()
            pltpu.sync_copy(acc.at[pl.ds(rbase, RPT)],
                            out.at[p, cid, pl.ds(rbase, RPT)])
            plsc.subcore_barrier()

    return pl.kernel(
        body,
        mesh=_mesh,
        out_type=jax.ShapeDtypeStruct((n_pass, NC, NP, 128), jnp.float32),
        scratch_types=[
            pltpu.VMEM_SHARED((NP, 128), jnp.float32),
            pltpu.VMEM((HCH, CH), jnp.int32),
            pltpu.VMEM((HCH, CH), jnp.int32),
            pltpu.VMEM((CH, 128), jnp.float32),
            pltpu.VMEM((CH, 128), jnp.float32),
            pltpu.SemaphoreType.DMA,
            pltpu.SemaphoreType.DMA,
        ],
    )


_seg4 = _make_seg_kernel(4, (0, 0, 1, 1))
_seg2 = _make_seg_kernel(2, (0, 1))

BN = 1000
NB = N // BN


BNP = 1024


def _dinv_body(dp_ref, dinv_ref):
    deg = dp_ref[0, 0, :, 0:1] + dp_ref[0, 1, :, 0:1] + 1.0
    dinv_ref[0] = lax.rsqrt(deg)


def _tc_dinv(degpart):
    return pl.pallas_call(
        _dinv_body,
        grid=(2, NP // BNP),
        in_specs=[pl.BlockSpec((1, 2, BNP, 128), lambda v, i: (v, 0, i, 0))],
        out_specs=pl.BlockSpec((1, BNP, 1), lambda v, i: (v, i, 0)),
        out_shape=jax.ShapeDtypeStruct((2, NP, 1), jnp.float32),
    )(degpart)


def _k2_body(x_ref, dv_ref, w0_ref, y_ref):
    dinv = dv_ref[0]
    xw = jnp.dot(x_ref[0], w0_ref[...], preferred_element_type=jnp.float32)
    y = xw * dinv
    y_ref[0, 0] = y[:, :128]
    y_ref[0, 1] = y[:, 128:]


def _k4_body(seg_ref, y_ref, dv_ref, w1_ref, b0_ref, y2_ref):
    dinv = dv_ref[0]
    hs = []
    for h in range(2):
        s = seg_ref[h, 0] + seg_ref[h, 1] + y_ref[0, h]
        hs.append(jnp.maximum(dinv * s + b0_ref[h], 0.0))
    hcat = jnp.concatenate(hs, axis=1)
    y2 = jnp.dot(hcat, w1_ref[...], preferred_element_type=jnp.float32)
    y2_ref[0] = y2 * dinv


def _k6_body(seg_ref, y2_ref, dv_ref, b1_ref, wi1_ref, bi1_ref, wi2_ref,
             bi2_ref, wc1_ref, bc1_ref, wc2_ref, bc2_ref, zi_ref, cp_ref):
    dinv = dv_ref[0]
    s = seg_ref[0, 0] + seg_ref[0, 1] + y2_ref[0]
    z = jnp.maximum(dinv * s + b1_ref[...], 0.0)
    hi = jnp.maximum(
        jnp.dot(z, wi1_ref[...], preferred_element_type=jnp.float32)
        + bi1_ref[...], 0.0)
    zi = (jnp.dot(hi, wi2_ref[...], preferred_element_type=jnp.float32)
          + bi2_ref[...])
    nrm = jnp.sqrt(jnp.sum(zi * zi, axis=1, keepdims=True))
    zi_ref[0] = zi / jnp.maximum(nrm, 1e-12)
    hc = jnp.maximum(
        jnp.dot(z, wc1_ref[...], preferred_element_type=jnp.float32)
        + bc1_ref[...], 0.0)
    lg = (jnp.dot(hc, wc2_ref[...], preferred_element_type=jnp.float32)
          + bc2_ref[...])
    col = lax.broadcasted_iota(jnp.int32, lg.shape, 1)
    lg = jnp.where(col < NCL, lg, -1e30)
    m = jnp.max(lg, axis=1, keepdims=True)
    e = jnp.exp(lg - m)
    cp_ref[0] = e / jnp.sum(e, axis=1, keepdims=True)


def _tc_scale_matmul(x, dinv, W0):
    return pl.pallas_call(
        _k2_body,
        grid=(2, NB),
        in_specs=[
            pl.BlockSpec((1, BN, 128), lambda v, i: (v, i, 0)),
            pl.BlockSpec((1, BN, 1), lambda v, i: (v, i, 0)),
            pl.BlockSpec((128, 256), lambda v, i: (0, 0)),
        ],
        out_specs=pl.BlockSpec((1, 2, BN, 128), lambda v, i: (v, 0, i, 0)),
        out_shape=jax.ShapeDtypeStruct((2, 2, N, 128), jnp.float32),
    )(x, dinv, W0)


def _tc_layer2_in(seg1, y1, dinv, W1, b0r):
    return pl.pallas_call(
        _k4_body,
        grid=(2, NB),
        in_specs=[
            pl.BlockSpec((2, 2, BN, 128), lambda v, i: (v, 0, i, 0)),
            pl.BlockSpec((1, 2, BN, 128), lambda v, i: (v, 0, i, 0)),
            pl.BlockSpec((1, BN, 1), lambda v, i: (v, i, 0)),
            pl.BlockSpec((256, 128), lambda v, i: (0, 0)),
            pl.BlockSpec((2, 128), lambda v, i: (0, 0)),
        ],
        out_specs=pl.BlockSpec((1, BN, 128), lambda v, i: (v, i, 0)),
        out_shape=jax.ShapeDtypeStruct((2, N, 128), jnp.float32),
    )(seg1, y1, dinv, W1, b0r)


def _tc_heads(seg2, y2, dinv, b1, Wi1, bi1, Wi2, bi2, Wc1, bc1, Wc2p, bc2p):
    wspec = pl.BlockSpec((128, 128), lambda v, i: (0, 0))
    bspec = pl.BlockSpec((128,), lambda v, i: (0,))
    return pl.pallas_call(
        _k6_body,
        grid=(2, NB),
        in_specs=[
            pl.BlockSpec((1, 2, BN, 128), lambda v, i: (v, 0, i, 0)),
            pl.BlockSpec((1, BN, 128), lambda v, i: (v, i, 0)),
            pl.BlockSpec((1, BN, 1), lambda v, i: (v, i, 0)),
            bspec, wspec, bspec, wspec, bspec, wspec, bspec, wspec, bspec,
        ],
        out_specs=[
            pl.BlockSpec((1, BN, 128), lambda v, i: (v, i, 0)),
            pl.BlockSpec((1, BN, 128), lambda v, i: (v, i, 0)),
        ],
        out_shape=[
            jax.ShapeDtypeStruct((2, N, 128), jnp.float32),
            jax.ShapeDtypeStruct((2, N, 128), jnp.float32),
        ],
    )(seg2, y2, dinv, b1, Wi1, bi1, Wi2, bi2, Wc1, bc1, Wc2p, bc2p)


def _prep_edges(edge_index):
    pad = NW * EPW - E
    pad_idx = jnp.arange(pad, dtype=jnp.int32)
    srcp = jnp.concatenate(
        [edge_index[0], pad_idx % N]).reshape(NW, NCH, CH)
    dstp = jnp.concatenate(
        [edge_index[1], N + pad_idx % (NP - N)]).reshape(NW, NCH, CH)
    return srcp, dstp


def kernel(x1, edge_index1, x2, edge_index2, W0, b0, W1, b1,
           Wi1, bi1, Wi2, bi2, Wc1, bc1, Wc2, bc2):
    src1, dst1 = _prep_edges(edge_index1)
    src2, dst2 = _prep_edges(edge_index2)
    zeros128 = jnp.zeros((ZR, 128), jnp.float32)
    ones128 = jnp.ones((CH, 128), jnp.float32)

    degpart = _deg_kernel(dst1, dst2, ones128, zeros128)
    dinv = _tc_dinv(degpart)

    x = jnp.stack([x1, x2])
    y1 = _tc_scale_matmul(x, dinv, W0)

    seg1 = _seg4(y1[0, 0], y1[0, 1], y1[1, 0], y1[1, 1],
                 src1, src2, dst1, dst2, zeros128)

    y2 = _tc_layer2_in(seg1, y1, dinv, W1, b0.reshape(2, 128))

    seg2 = _seg2(y2[0], y2[1], src1, src2, dst1, dst2, zeros128)

    Wc2p = jnp.pad(Wc2, ((0, 0), (0, 128 - NCL)))
    bc2p = jnp.pad(bc2, (0, 128 - NCL))
    zi, cp = _tc_heads(seg2, y2, dinv, b1, Wi1, bi1, Wi2, bi2,
                       Wc1, bc1, Wc2p, bc2p)

    return (zi[0], zi[1], cp[0, :, :NCL], cp[1, :, :NCL])


# speedup vs baseline: 1.1364x; 1.1364x over previous
import functools

import jax
import jax.numpy as jnp
from jax import lax
from jax.experimental import pallas as pl
from jax.experimental.pallas import tpu as pltpu
from jax.experimental.pallas import tpu_sc as plsc

N = 10000
E = 320000
NCL = 3

NC = 2
NS = 16
NW = NC * NS
CH = 128
NCH = 80
EPW = NCH * CH
HCH = NCH // 2
NP = 10240
RPT = NP // NS
ZR = 128

_mesh = plsc.VectorSubcoreMesh(core_axis_name="c", subcore_axis_name="s")


def _sc_worker_ids():
    cid = lax.axis_index("c")
    sid = lax.axis_index("s")
    wid = sid * NC + cid
    return cid, sid, wid


@functools.partial(
    pl.kernel,
    mesh=_mesh,
    out_type=jax.ShapeDtypeStruct((2 * NC * NP,), jnp.float32),
    scratch_types=[
        pltpu.VMEM_SHARED((NP,), jnp.float32),
        pltpu.VMEM((NCH, CH), jnp.int32),
        pltpu.VMEM((CH,), jnp.float32),
        pltpu.VMEM((RPT,), jnp.float32),
    ],
)
def _deg_kernel(dst1, dst2, out, acc, dall, ones_v, zeros_v):
    cid, sid, wid = _sc_worker_ids()
    rbase = sid * RPT
    for j in range(CH // 16):
        ones_v[pl.ds(j * 16, 16)] = jnp.full((16,), 1.0, jnp.float32)
    for j in range(RPT // 16):
        zeros_v[pl.ds(j * 16, 16)] = jnp.zeros((16,), jnp.float32)
    for v, dref in ((0, dst1), (1, dst2)):
        pltpu.sync_copy(zeros_v, acc.at[pl.ds(rbase, RPT)])
        pltpu.sync_copy(dref.at[wid], dall)
        plsc.subcore_barrier()

        def chunk(i, carry):
            pltpu.sync_copy(ones_v, acc.at[dall.at[i]], add=True)
            return carry

        lax.fori_loop(0, NCH, chunk, 0)
        plsc.subcore_barrier()
        pltpu.sync_copy(acc.at[pl.ds(rbase, RPT)],
                        out.at[pl.ds((v * NC + cid) * NP + rbase, RPT)])
        plsc.subcore_barrier()


def _make_seg_kernel(n_pass, view_of_pass):
    def body(*refs):
        ys = refs[:n_pass]
        srcs = refs[n_pass:n_pass + 2]
        dsts = refs[n_pass + 2:n_pass + 4]
        zeros_h = refs[n_pass + 4]
        out = refs[n_pass + 5]
        acc, sall, dall, rows0, rows1, sem0, sem1 = refs[n_pass + 6:]
        cid, sid, wid = _sc_worker_ids()
        rbase = sid * RPT
        for p in range(n_pass):
            v = view_of_pass[p]
            yref = ys[p]
            for j in range(RPT // ZR):
                pltpu.sync_copy(zeros_h, acc.at[pl.ds(rbase + j * ZR, ZR)])
            plsc.subcore_barrier()

            for half in range(2):
                cbase = half * HCH
                pltpu.sync_copy(srcs[v].at[wid, pl.ds(cbase, HCH)], sall)
                pltpu.sync_copy(dsts[v].at[wid, pl.ds(cbase, HCH)], dall)

                pltpu.async_copy(yref.at[sall.at[0]], rows0, sem0)

                def pair(t, carry, yref=yref):
                    c0 = 2 * t
                    pltpu.async_copy(yref.at[sall.at[c0 + 1]], rows1, sem1)
                    pltpu.make_async_copy(yref.at[sall.at[c0]], rows0,
                                          sem0).wait()
                    pltpu.sync_copy(rows0, acc.at[dall.at[c0]], add=True)
                    pltpu.async_copy(yref.at[sall.at[c0 + 2]], rows0, sem0)
                    pltpu.make_async_copy(yref.at[sall.at[c0 + 1]], rows1,
                                          sem1).wait()
                    pltpu.sync_copy(rows1, acc.at[dall.at[c0 + 1]], add=True)
                    return carry

                lax.fori_loop(0, HCH // 2 - 1, pair, 0)
                cl = HCH - 2
                pltpu.async_copy(yref.at[sall.at[cl + 1]], rows1, sem1)
                pltpu.make_async_copy(yref.at[sall.at[cl]], rows0, sem0).wait()
                pltpu.sync_copy(rows0, acc.at[dall.at[cl]], add=True)
                pltpu.make_async_copy(yref.at[sall.at[cl + 1]], rows1,
                                      sem1).wait()
                pltpu.sync_copy(rows1, acc.at[dall.at[cl + 1]], add=True)

            plsc.subcore_barrier()
            pltpu.sync_copy(acc.at[pl.ds(rbase, RPT)],
                            out.at[p, cid, pl.ds(rbase, RPT)])
            plsc.subcore_barrier()

    return pl.kernel(
        body,
        mesh=_mesh,
        out_type=jax.ShapeDtypeStruct((n_pass, NC, NP, 128), jnp.float32),
        scratch_types=[
            pltpu.VMEM_SHARED((NP, 128), jnp.float32),
            pltpu.VMEM((HCH, CH), jnp.int32),
            pltpu.VMEM((HCH, CH), jnp.int32),
            pltpu.VMEM((CH, 128), jnp.float32),
            pltpu.VMEM((CH, 128), jnp.float32),
            pltpu.SemaphoreType.DMA,
            pltpu.SemaphoreType.DMA,
        ],
    )


_seg4 = _make_seg_kernel(4, (0, 0, 1, 1))
_seg2 = _make_seg_kernel(2, (0, 1))

BN = 1000
NB = N // BN


BNP = 1024


def _dinv_body(dp_ref, dinv_ref):
    deg = dp_ref[0, 0] + dp_ref[0, 1] + 1.0
    dinv_ref[0] = lax.rsqrt(deg)


def _tc_dinv(degpart):
    return pl.pallas_call(
        _dinv_body,
        grid=(2, NP // BNP),
        in_specs=[pl.BlockSpec((1, 2, BNP, 1), lambda v, i: (v, 0, i, 0))],
        out_specs=pl.BlockSpec((1, BNP, 1), lambda v, i: (v, i, 0)),
        out_shape=jax.ShapeDtypeStruct((2, NP, 1), jnp.float32),
    )(degpart)


def _k2_body(x_ref, dv_ref, w0_ref, y_ref):
    dinv = dv_ref[0]
    xw = jnp.dot(x_ref[0], w0_ref[...], preferred_element_type=jnp.float32)
    y = xw * dinv
    y_ref[0, 0] = y[:, :128]
    y_ref[0, 1] = y[:, 128:]


def _k4_body(seg_ref, y_ref, dv_ref, w1_ref, b0_ref, y2_ref):
    dinv = dv_ref[0]
    hs = []
    for h in range(2):
        s = seg_ref[h, 0] + seg_ref[h, 1] + y_ref[0, h]
        hs.append(jnp.maximum(dinv * s + b0_ref[h], 0.0))
    hcat = jnp.concatenate(hs, axis=1)
    y2 = jnp.dot(hcat, w1_ref[...], preferred_element_type=jnp.float32)
    y2_ref[0] = y2 * dinv


def _k6_body(seg_ref, y2_ref, dv_ref, b1_ref, wi1_ref, bi1_ref, wi2_ref,
             bi2_ref, wc1_ref, bc1_ref, wc2_ref, bc2_ref, zi_ref, cp_ref):
    dinv = dv_ref[0]
    s = seg_ref[0, 0] + seg_ref[0, 1] + y2_ref[0]
    z = jnp.maximum(dinv * s + b1_ref[...], 0.0)
    hi = jnp.maximum(
        jnp.dot(z, wi1_ref[...], preferred_element_type=jnp.float32)
        + bi1_ref[...], 0.0)
    zi = (jnp.dot(hi, wi2_ref[...], preferred_element_type=jnp.float32)
          + bi2_ref[...])
    nrm = jnp.sqrt(jnp.sum(zi * zi, axis=1, keepdims=True))
    zi_ref[0] = zi / jnp.maximum(nrm, 1e-12)
    hc = jnp.maximum(
        jnp.dot(z, wc1_ref[...], preferred_element_type=jnp.float32)
        + bc1_ref[...], 0.0)
    lg = (jnp.dot(hc, wc2_ref[...], preferred_element_type=jnp.float32)
          + bc2_ref[...])
    col = lax.broadcasted_iota(jnp.int32, lg.shape, 1)
    lg = jnp.where(col < NCL, lg, -1e30)
    m = jnp.max(lg, axis=1, keepdims=True)
    e = jnp.exp(lg - m)
    cp_ref[0] = e / jnp.sum(e, axis=1, keepdims=True)


def _tc_scale_matmul(x, dinv, W0):
    return pl.pallas_call(
        _k2_body,
        grid=(2, NB),
        in_specs=[
            pl.BlockSpec((1, BN, 128), lambda v, i: (v, i, 0)),
            pl.BlockSpec((1, BN, 1), lambda v, i: (v, i, 0)),
            pl.BlockSpec((128, 256), lambda v, i: (0, 0)),
        ],
        out_specs=pl.BlockSpec((1, 2, BN, 128), lambda v, i: (v, 0, i, 0)),
        out_shape=jax.ShapeDtypeStruct((2, 2, N, 128), jnp.float32),
    )(x, dinv, W0)


def _tc_layer2_in(seg1, y1, dinv, W1, b0r):
    return pl.pallas_call(
        _k4_body,
        grid=(2, NB),
        in_specs=[
            pl.BlockSpec((2, 2, BN, 128), lambda v, i: (v, 0, i, 0)),
            pl.BlockSpec((1, 2, BN, 128), lambda v, i: (v, 0, i, 0)),
            pl.BlockSpec((1, BN, 1), lambda v, i: (v, i, 0)),
            pl.BlockSpec((256, 128), lambda v, i: (0, 0)),
            pl.BlockSpec((2, 128), lambda v, i: (0, 0)),
        ],
        out_specs=pl.BlockSpec((1, BN, 128), lambda v, i: (v, i, 0)),
        out_shape=jax.ShapeDtypeStruct((2, N, 128), jnp.float32),
    )(seg1, y1, dinv, W1, b0r)


def _tc_heads(seg2, y2, dinv, b1, Wi1, bi1, Wi2, bi2, Wc1, bc1, Wc2p, bc2p):
    wspec = pl.BlockSpec((128, 128), lambda v, i: (0, 0))
    bspec = pl.BlockSpec((128,), lambda v, i: (0,))
    return pl.pallas_call(
        _k6_body,
        grid=(2, NB),
        in_specs=[
            pl.BlockSpec((1, 2, BN, 128), lambda v, i: (v, 0, i, 0)),
            pl.BlockSpec((1, BN, 128), lambda v, i: (v, i, 0)),
            pl.BlockSpec((1, BN, 1), lambda v, i: (v, i, 0)),
            bspec, wspec, bspec, wspec, bspec, wspec, bspec, wspec, bspec,
        ],
        out_specs=[
            pl.BlockSpec((1, BN, 128), lambda v, i: (v, i, 0)),
            pl.BlockSpec((1, BN, 128), lambda v, i: (v, i, 0)),
        ],
        out_shape=[
            jax.ShapeDtypeStruct((2, N, 128), jnp.float32),
            jax.ShapeDtypeStruct((2, N, 128), jnp.float32),
        ],
    )(seg2, y2, dinv, b1, Wi1, bi1, Wi2, bi2, Wc1, bc1, Wc2p, bc2p)


def _prep_edges(edge_index):
    pad = NW * EPW - E
    pad_idx = jnp.arange(pad, dtype=jnp.int32)
    srcp = jnp.concatenate(
        [edge_index[0], pad_idx % N]).reshape(NW, NCH, CH)
    dstp = jnp.concatenate(
        [edge_index[1], N + pad_idx % (NP - N)]).reshape(NW, NCH, CH)
    return srcp, dstp


def kernel(x1, edge_index1, x2, edge_index2, W0, b0, W1, b1,
           Wi1, bi1, Wi2, bi2, Wc1, bc1, Wc2, bc2):
    src1, dst1 = _prep_edges(edge_index1)
    src2, dst2 = _prep_edges(edge_index2)
    zeros128 = jnp.zeros((ZR, 128), jnp.float32)

    degpart = _deg_kernel(dst1, dst2).reshape(2, NC, NP, 1)
    dinv = _tc_dinv(degpart)

    x = jnp.stack([x1, x2])
    y1 = _tc_scale_matmul(x, dinv, W0)

    seg1 = _seg4(y1[0, 0], y1[0, 1], y1[1, 0], y1[1, 1],
                 src1, src2, dst1, dst2, zeros128)

    y2 = _tc_layer2_in(seg1, y1, dinv, W1, b0.reshape(2, 128))

    seg2 = _seg2(y2[0], y2[1], src1, src2, dst1, dst2, zeros128)

    Wc2p = jnp.pad(Wc2, ((0, 0), (0, 128 - NCL)))
    bc2p = jnp.pad(bc2, (0, 128 - NCL))
    zi, cp = _tc_heads(seg2, y2, dinv, b1, Wi1, bi1, Wi2, bi2,
                       Wc1, bc1, Wc2p, bc2p)

    return (zi[0], zi[1], cp[0, :, :NCL], cp[1, :, :NCL])


# speedup vs baseline: 1.1568x; 1.0179x over previous
import functools

import jax
import jax.numpy as jnp
from jax import lax
from jax.experimental import pallas as pl
from jax.experimental.pallas import tpu as pltpu
from jax.experimental.pallas import tpu_sc as plsc

N = 10000
E = 320000
NCL = 3

NC = 2
NS = 16
NW = NC * NS
CH = 128
NCH = 80
EPW = NCH * CH
HCH = NCH // 2
NP = 10240
RPT = NP // NS
ZR = 128

_mesh = plsc.VectorSubcoreMesh(core_axis_name="c", subcore_axis_name="s")


def _sc_worker_ids():
    cid = lax.axis_index("c")
    sid = lax.axis_index("s")
    wid = sid * NC + cid
    return cid, sid, wid


@functools.partial(
    pl.kernel,
    mesh=_mesh,
    out_type=jax.ShapeDtypeStruct((2 * NC * NP,), jnp.float32),
    scratch_types=[
        pltpu.VMEM_SHARED((NP,), jnp.float32),
        pltpu.VMEM((NCH, CH), jnp.int32),
        pltpu.VMEM((CH,), jnp.float32),
        pltpu.VMEM((RPT,), jnp.float32),
    ],
)
def _deg_kernel(dst1, dst2, out, acc, dall, ones_v, zeros_v):
    cid, sid, wid = _sc_worker_ids()
    rbase = sid * RPT
    for j in range(CH // 16):
        ones_v[pl.ds(j * 16, 16)] = jnp.full((16,), 1.0, jnp.float32)
    for j in range(RPT // 16):
        zeros_v[pl.ds(j * 16, 16)] = jnp.zeros((16,), jnp.float32)
    for v, dref in ((0, dst1), (1, dst2)):
        pltpu.sync_copy(zeros_v, acc.at[pl.ds(rbase, RPT)])
        pltpu.sync_copy(dref.at[wid], dall)
        plsc.subcore_barrier()

        def chunk(i, carry):
            pltpu.sync_copy(ones_v, acc.at[dall.at[i]], add=True)
            return carry

        lax.fori_loop(0, NCH, chunk, 0)
        plsc.subcore_barrier()
        pltpu.sync_copy(acc.at[pl.ds(rbase, RPT)],
                        out.at[pl.ds((v * NC + cid) * NP + rbase, RPT)])
        plsc.subcore_barrier()


def _make_seg_kernel(n_pass, view_of_pass):
    def body(*refs):
        ys = refs[:n_pass]
        srcs = refs[n_pass:n_pass + 2]
        dsts = refs[n_pass + 2:n_pass + 4]
        zeros_h = refs[n_pass + 4]
        out = refs[n_pass + 5]
        acc, sall, dall, rows0, rows1, sem0, sem1 = refs[n_pass + 6:]
        cid, sid, wid = _sc_worker_ids()
        rbase = sid * RPT
        for p in range(n_pass):
            v = view_of_pass[p]
            yref = ys[p]
            for j in range(RPT // ZR):
                pltpu.sync_copy(zeros_h, acc.at[pl.ds(rbase + j * ZR, ZR)])
            plsc.subcore_barrier()

            for half in range(2):
                cbase = half * HCH
                pltpu.sync_copy(srcs[v].at[wid, pl.ds(cbase, HCH)], sall)
                pltpu.sync_copy(dsts[v].at[wid, pl.ds(cbase, HCH)], dall)

                pltpu.async_copy(yref.at[sall.at[0]], rows0, sem0)

                def pair(t, carry, yref=yref):
                    c0 = 2 * t
                    pltpu.async_copy(yref.at[sall.at[c0 + 1]], rows1, sem1)
                    pltpu.make_async_copy(yref.at[sall.at[c0]], rows0,
                                          sem0).wait()
                    pltpu.sync_copy(rows0, acc.at[dall.at[c0]], add=True)
                    pltpu.async_copy(yref.at[sall.at[c0 + 2]], rows0, sem0)
                    pltpu.make_async_copy(yref.at[sall.at[c0 + 1]], rows1,
                                          sem1).wait()
                    pltpu.sync_copy(rows1, acc.at[dall.at[c0 + 1]], add=True)
                    return carry

                lax.fori_loop(0, HCH // 2 - 1, pair, 0)
                cl = HCH - 2
                pltpu.async_copy(yref.at[sall.at[cl + 1]], rows1, sem1)
                pltpu.make_async_copy(yref.at[sall.at[cl]], rows0, sem0).wait()
                pltpu.sync_copy(rows0, acc.at[dall.at[cl]], add=True)
                pltpu.make_async_copy(yref.at[sall.at[cl + 1]], rows1,
                                      sem1).wait()
                pltpu.sync_copy(rows1, acc.at[dall.at[cl + 1]], add=True)

            plsc.subcore_barrier()
            pltpu.sync_copy(acc.at[pl.ds(rbase, RPT)],
                            out.at[p, cid, pl.ds(rbase, RPT)])
            plsc.subcore_barrier()

    return pl.kernel(
        body,
        mesh=_mesh,
        out_type=jax.ShapeDtypeStruct((n_pass, NC, NP, 128), jnp.float32),
        scratch_types=[
            pltpu.VMEM_SHARED((NP, 128), jnp.float32),
            pltpu.VMEM((HCH, CH), jnp.int32),
            pltpu.VMEM((HCH, CH), jnp.int32),
            pltpu.VMEM((CH, 128), jnp.float32),
            pltpu.VMEM((CH, 128), jnp.float32),
            pltpu.SemaphoreType.DMA,
            pltpu.SemaphoreType.DMA,
        ],
    )


_segL1 = _make_seg_kernel(2, (0, 0))
_segL2 = _make_seg_kernel(1, (0,))

BN = 1000
NB = N // BN


BNP = 1024


def _dinv_body(dp_ref, dinv_ref):
    deg = dp_ref[0, 0] + dp_ref[0, 1] + 1.0
    dinv_ref[0] = lax.rsqrt(deg)


def _tc_dinv(degpart):
    return pl.pallas_call(
        _dinv_body,
        grid=(2, NP // BNP),
        in_specs=[pl.BlockSpec((1, 2, BNP, 1), lambda v, i: (v, 0, i, 0))],
        out_specs=pl.BlockSpec((1, BNP, 1), lambda v, i: (v, i, 0)),
        out_shape=jax.ShapeDtypeStruct((2, NP, 1), jnp.float32),
    )(degpart)


def _k2_body(x_ref, dv_ref, w0_ref, y_ref):
    dinv = dv_ref[0]
    xw = jnp.dot(x_ref[0], w0_ref[...], preferred_element_type=jnp.float32)
    y = xw * dinv
    y_ref[0, 0] = y[:, :128]
    y_ref[0, 1] = y[:, 128:]


def _k4_body(seg_ref, y_ref, dv_ref, w1_ref, b0_ref, y2_ref):
    dinv = dv_ref[0]
    hs = []
    for h in range(2):
        s = seg_ref[h, 0] + seg_ref[h, 1] + y_ref[0, h]
        hs.append(jnp.maximum(dinv * s + b0_ref[h], 0.0))
    hcat = jnp.concatenate(hs, axis=1)
    y2 = jnp.dot(hcat, w1_ref[...], preferred_element_type=jnp.float32)
    y2_ref[0] = y2 * dinv


def _k6_body(seg_ref, y2_ref, dv_ref, b1_ref, wi1_ref, bi1_ref, wi2_ref,
             bi2_ref, wc1_ref, bc1_ref, wc2_ref, bc2_ref, zi_ref, cp_ref):
    dinv = dv_ref[0]
    s = seg_ref[0, 0] + seg_ref[0, 1] + y2_ref[0]
    z = jnp.maximum(dinv * s + b1_ref[...], 0.0)
    hi = jnp.maximum(
        jnp.dot(z, wi1_ref[...], preferred_element_type=jnp.float32)
        + bi1_ref[...], 0.0)
    zi = (jnp.dot(hi, wi2_ref[...], preferred_element_type=jnp.float32)
          + bi2_ref[...])
    nrm = jnp.sqrt(jnp.sum(zi * zi, axis=1, keepdims=True))
    zi_ref[0] = zi / jnp.maximum(nrm, 1e-12)
    hc = jnp.maximum(
        jnp.dot(z, wc1_ref[...], preferred_element_type=jnp.float32)
        + bc1_ref[...], 0.0)
    lg = (jnp.dot(hc, wc2_ref[...], preferred_element_type=jnp.float32)
          + bc2_ref[...])
    col = lax.broadcasted_iota(jnp.int32, lg.shape, 1)
    lg = jnp.where(col < NCL, lg, -1e30)
    m = jnp.max(lg, axis=1, keepdims=True)
    e = jnp.exp(lg - m)
    cp_ref[0] = e / jnp.sum(e, axis=1, keepdims=True)


def _tc_scale_matmul(x, dinv, W0):
    return pl.pallas_call(
        _k2_body,
        grid=(2, NB),
        in_specs=[
            pl.BlockSpec((1, BN, 128), lambda v, i: (v, i, 0)),
            pl.BlockSpec((1, BN, 1), lambda v, i: (v, i, 0)),
            pl.BlockSpec((128, 256), lambda v, i: (0, 0)),
        ],
        out_specs=pl.BlockSpec((1, 2, BN, 128), lambda v, i: (v, 0, i, 0)),
        out_shape=jax.ShapeDtypeStruct((2, 2, N, 128), jnp.float32),
    )(x, dinv, W0)


def _tc_layer2_in(seg1, y1, dinv, W1, b0r):
    nv = y1.shape[0]
    return pl.pallas_call(
        _k4_body,
        grid=(nv, NB),
        in_specs=[
            pl.BlockSpec((2, 2, BN, 128), lambda v, i: (v, 0, i, 0)),
            pl.BlockSpec((1, 2, BN, 128), lambda v, i: (v, 0, i, 0)),
            pl.BlockSpec((1, BN, 1), lambda v, i: (v, i, 0)),
            pl.BlockSpec((256, 128), lambda v, i: (0, 0)),
            pl.BlockSpec((2, 128), lambda v, i: (0, 0)),
        ],
        out_specs=pl.BlockSpec((1, BN, 128), lambda v, i: (v, i, 0)),
        out_shape=jax.ShapeDtypeStruct((nv, N, 128), jnp.float32),
    )(seg1, y1, dinv, W1, b0r)


def _tc_heads(seg2, y2, dinv, b1, Wi1, bi1, Wi2, bi2, Wc1, bc1, Wc2p, bc2p):
    nv = y2.shape[0]
    wspec = pl.BlockSpec((128, 128), lambda v, i: (0, 0))
    bspec = pl.BlockSpec((128,), lambda v, i: (0,))
    return pl.pallas_call(
        _k6_body,
        grid=(nv, NB),
        in_specs=[
            pl.BlockSpec((1, 2, BN, 128), lambda v, i: (v, 0, i, 0)),
            pl.BlockSpec((1, BN, 128), lambda v, i: (v, i, 0)),
            pl.BlockSpec((1, BN, 1), lambda v, i: (v, i, 0)),
            bspec, wspec, bspec, wspec, bspec, wspec, bspec, wspec, bspec,
        ],
        out_specs=[
            pl.BlockSpec((1, BN, 128), lambda v, i: (v, i, 0)),
            pl.BlockSpec((1, BN, 128), lambda v, i: (v, i, 0)),
        ],
        out_shape=[
            jax.ShapeDtypeStruct((nv, N, 128), jnp.float32),
            jax.ShapeDtypeStruct((nv, N, 128), jnp.float32),
        ],
    )(seg2, y2, dinv, b1, Wi1, bi1, Wi2, bi2, Wc1, bc1, Wc2p, bc2p)


def _prep_edges(edge_index):
    pad = NW * EPW - E
    pad_idx = jnp.arange(pad, dtype=jnp.int32)
    srcp = jnp.concatenate(
        [edge_index[0], pad_idx % N]).reshape(NW, NCH, CH)
    dstp = jnp.concatenate(
        [edge_index[1], N + pad_idx % (NP - N)]).reshape(NW, NCH, CH)
    return srcp, dstp


def kernel(x1, edge_index1, x2, edge_index2, W0, b0, W1, b1,
           Wi1, bi1, Wi2, bi2, Wc1, bc1, Wc2, bc2):
    src1, dst1 = _prep_edges(edge_index1)
    src2, dst2 = _prep_edges(edge_index2)
    zeros128 = jnp.zeros((ZR, 128), jnp.float32)

    degpart = _deg_kernel(dst1, dst2).reshape(2, NC, NP, 1)
    dinv = _tc_dinv(degpart)

    x = jnp.stack([x1, x2])
    y1 = _tc_scale_matmul(x, dinv, W0)
    b0r = b0.reshape(2, 128)
    Wc2p = jnp.pad(Wc2, ((0, 0), (0, 128 - NCL)))
    bc2p = jnp.pad(bc2, (0, 128 - NCL))

    outs = []
    for v, (sv, dv) in enumerate(((src1, dst1), (src2, dst2))):
        s1 = _segL1(y1[v, 0], y1[v, 1], sv, sv, dv, dv, zeros128)
        y2 = _tc_layer2_in(s1, y1[v:v + 1], dinv[v:v + 1], W1, b0r)
        s2 = _segL2(y2[0], sv, sv, dv, dv, zeros128)
        zi, cp = _tc_heads(s2, y2, dinv[v:v + 1], b1, Wi1, bi1, Wi2, bi2,
                           Wc1, bc1, Wc2p, bc2p)
        outs.append((zi[0], cp[0, :, :NCL]))

    return (outs[0][0], outs[1][0], outs[0][1], outs[1][1])


# speedup vs baseline: 1.2310x; 1.0642x over previous
import functools

import jax
import jax.numpy as jnp
from jax import lax
from jax.experimental import pallas as pl
from jax.experimental.pallas import tpu as pltpu
from jax.experimental.pallas import tpu_sc as plsc

N = 10000
E = 320000
NCL = 3

NC = 2
NS = 16
NW = NC * NS
CH = 128
NCH = 80
EPW = NCH * CH
HCH = NCH // 2
NP = 10240
RPT = NP // NS
ZR = 640

_mesh = plsc.VectorSubcoreMesh(core_axis_name="c", subcore_axis_name="s")


def _sc_worker_ids():
    cid = lax.axis_index("c")
    sid = lax.axis_index("s")
    wid = sid * NC + cid
    return cid, sid, wid


@functools.partial(
    pl.kernel,
    mesh=_mesh,
    out_type=jax.ShapeDtypeStruct((2 * NC * NP,), jnp.float32),
    scratch_types=[
        pltpu.VMEM_SHARED((NP,), jnp.float32),
        pltpu.VMEM((NCH, CH), jnp.int32),
        pltpu.VMEM((CH,), jnp.float32),
        pltpu.VMEM((RPT,), jnp.float32),
    ],
)
def _deg_kernel(dst1, dst2, out, acc, dall, ones_v, zeros_v):
    cid, sid, wid = _sc_worker_ids()
    rbase = sid * RPT
    for j in range(CH // 16):
        ones_v[pl.ds(j * 16, 16)] = jnp.full((16,), 1.0, jnp.float32)
    for j in range(RPT // 16):
        zeros_v[pl.ds(j * 16, 16)] = jnp.zeros((16,), jnp.float32)
    for v, dref in ((0, dst1), (1, dst2)):
        pltpu.sync_copy(zeros_v, acc.at[pl.ds(rbase, RPT)])
        pltpu.sync_copy(dref.at[wid], dall)
        plsc.subcore_barrier()

        def chunk(i, carry):
            pltpu.sync_copy(ones_v, acc.at[dall.at[i]], add=True)
            return carry

        lax.fori_loop(0, NCH, chunk, 0)
        plsc.subcore_barrier()
        pltpu.sync_copy(acc.at[pl.ds(rbase, RPT)],
                        out.at[pl.ds((v * NC + cid) * NP + rbase, RPT)])
        plsc.subcore_barrier()


def _make_seg_kernel(n_pass, view_of_pass):
    def body(*refs):
        ys = refs[:n_pass]
        srcs = refs[n_pass:n_pass + 2]
        dsts = refs[n_pass + 2:n_pass + 4]
        zeros_h = refs[n_pass + 4]
        out = refs[n_pass + 5]
        acc, sall, dall, rows0, rows1, sem0, sem1 = refs[n_pass + 6:]
        cid, sid, wid = _sc_worker_ids()
        rbase = sid * RPT
        for p in range(n_pass):
            v = view_of_pass[p]
            yref = ys[p]
            for j in range(RPT // ZR):
                pltpu.sync_copy(zeros_h, acc.at[pl.ds(rbase + j * ZR, ZR)])
            plsc.subcore_barrier()

            for half in range(2):
                cbase = half * HCH
                pltpu.sync_copy(srcs[v].at[wid, pl.ds(cbase, HCH)], sall)
                pltpu.sync_copy(dsts[v].at[wid, pl.ds(cbase, HCH)], dall)

                pltpu.async_copy(yref.at[sall.at[0]], rows0, sem0)

                def pair(t, carry, yref=yref):
                    c0 = 2 * t
                    pltpu.async_copy(yref.at[sall.at[c0 + 1]], rows1, sem1)
                    pltpu.make_async_copy(yref.at[sall.at[c0]], rows0,
                                          sem0).wait()
                    pltpu.sync_copy(rows0, acc.at[dall.at[c0]], add=True)
                    pltpu.async_copy(yref.at[sall.at[c0 + 2]], rows0, sem0)
                    pltpu.make_async_copy(yref.at[sall.at[c0 + 1]], rows1,
                                          sem1).wait()
                    pltpu.sync_copy(rows1, acc.at[dall.at[c0 + 1]], add=True)
                    return carry

                lax.fori_loop(0, HCH // 2 - 1, pair, 0)
                cl = HCH - 2
                pltpu.async_copy(yref.at[sall.at[cl + 1]], rows1, sem1)
                pltpu.make_async_copy(yref.at[sall.at[cl]], rows0, sem0).wait()
                pltpu.sync_copy(rows0, acc.at[dall.at[cl]], add=True)
                pltpu.make_async_copy(yref.at[sall.at[cl + 1]], rows1,
                                      sem1).wait()
                pltpu.sync_copy(rows1, acc.at[dall.at[cl + 1]], add=True)

            plsc.subcore_barrier()
            pltpu.sync_copy(acc.at[pl.ds(rbase, RPT)],
                            out.at[p, cid, pl.ds(rbase, RPT)])
            plsc.subcore_barrier()

    return pl.kernel(
        body,
        mesh=_mesh,
        out_type=jax.ShapeDtypeStruct((n_pass, NC, NP, 128), jnp.float32),
        scratch_types=[
            pltpu.VMEM_SHARED((NP, 128), jnp.float32),
            pltpu.VMEM((HCH, CH), jnp.int32),
            pltpu.VMEM((HCH, CH), jnp.int32),
            pltpu.VMEM((CH, 128), jnp.float32),
            pltpu.VMEM((CH, 128), jnp.float32),
            pltpu.SemaphoreType.DMA,
            pltpu.SemaphoreType.DMA,
        ],
    )


_segL1 = _make_seg_kernel(2, (0, 0))
_segL2 = _make_seg_kernel(1, (0,))

BN = 2000
NB = N // BN


BNP = 1024


def _dinv_body(dp_ref, dinv_ref):
    deg = dp_ref[0, 0] + dp_ref[0, 1] + 1.0
    dinv_ref[0] = lax.rsqrt(deg)


def _tc_dinv(degpart):
    return pl.pallas_call(
        _dinv_body,
        grid=(2, NP // BNP),
        in_specs=[pl.BlockSpec((1, 2, BNP, 1), lambda v, i: (v, 0, i, 0))],
        out_specs=pl.BlockSpec((1, BNP, 1), lambda v, i: (v, i, 0)),
        out_shape=jax.ShapeDtypeStruct((2, NP, 1), jnp.float32),
    )(degpart)


def _k2_body(x_ref, dv_ref, w0_ref, y_ref):
    dinv = dv_ref[0]
    xw = jnp.dot(x_ref[0], w0_ref[...], preferred_element_type=jnp.float32)
    y = xw * dinv
    y_ref[0, 0] = y[:, :128]
    y_ref[0, 1] = y[:, 128:]


def _k4_body(seg_ref, y_ref, dv_ref, w1_ref, b0_ref, y2_ref):
    dinv = dv_ref[0]
    hs = []
    for h in range(2):
        s = seg_ref[h, 0] + seg_ref[h, 1] + y_ref[0, h]
        hs.append(jnp.maximum(dinv * s + b0_ref[h], 0.0))
    hcat = jnp.concatenate(hs, axis=1)
    y2 = jnp.dot(hcat, w1_ref[...], preferred_element_type=jnp.float32)
    y2_ref[0] = y2 * dinv


def _k6_body(seg_ref, y2_ref, dv_ref, b1_ref, wi1_ref, bi1_ref, wi2_ref,
             bi2_ref, wc1_ref, bc1_ref, wc2_ref, bc2_ref, zi_ref, cp_ref):
    dinv = dv_ref[0]
    s = seg_ref[0, 0] + seg_ref[0, 1] + y2_ref[0]
    z = jnp.maximum(dinv * s + b1_ref[...], 0.0)
    hi = jnp.maximum(
        jnp.dot(z, wi1_ref[...], preferred_element_type=jnp.float32)
        + bi1_ref[...], 0.0)
    zi = (jnp.dot(hi, wi2_ref[...], preferred_element_type=jnp.float32)
          + bi2_ref[...])
    nrm = jnp.sqrt(jnp.sum(zi * zi, axis=1, keepdims=True))
    zi_ref[0] = zi / jnp.maximum(nrm, 1e-12)
    hc = jnp.maximum(
        jnp.dot(z, wc1_ref[...], preferred_element_type=jnp.float32)
        + bc1_ref[...], 0.0)
    lg = (jnp.dot(hc, wc2_ref[...], preferred_element_type=jnp.float32)
          + bc2_ref[...])
    col = lax.broadcasted_iota(jnp.int32, lg.shape, 1)
    lg = jnp.where(col < NCL, lg, -1e30)
    m = jnp.max(lg, axis=1, keepdims=True)
    e = jnp.exp(lg - m)
    cp_ref[0] = e / jnp.sum(e, axis=1, keepdims=True)


def _tc_scale_matmul(x, dinv, W0):
    return pl.pallas_call(
        _k2_body,
        grid=(2, NB),
        in_specs=[
            pl.BlockSpec((1, BN, 128), lambda v, i: (v, i, 0)),
            pl.BlockSpec((1, BN, 1), lambda v, i: (v, i, 0)),
            pl.BlockSpec((128, 256), lambda v, i: (0, 0)),
        ],
        out_specs=pl.BlockSpec((1, 2, BN, 128), lambda v, i: (v, 0, i, 0)),
        out_shape=jax.ShapeDtypeStruct((2, 2, N, 128), jnp.float32),
    )(x, dinv, W0)


def _tc_layer2_in(seg1, y1, dinv, W1, b0r):
    nv = y1.shape[0]
    return pl.pallas_call(
        _k4_body,
        grid=(nv, NB),
        in_specs=[
            pl.BlockSpec((2, 2, BN, 128), lambda v, i: (v, 0, i, 0)),
            pl.BlockSpec((1, 2, BN, 128), lambda v, i: (v, 0, i, 0)),
            pl.BlockSpec((1, BN, 1), lambda v, i: (v, i, 0)),
            pl.BlockSpec((256, 128), lambda v, i: (0, 0)),
            pl.BlockSpec((2, 128), lambda v, i: (0, 0)),
        ],
        out_specs=pl.BlockSpec((1, BN, 128), lambda v, i: (v, i, 0)),
        out_shape=jax.ShapeDtypeStruct((nv, N, 128), jnp.float32),
    )(seg1, y1, dinv, W1, b0r)


def _tc_heads(seg2, y2, dinv, b1, Wi1, bi1, Wi2, bi2, Wc1, bc1, Wc2p, bc2p):
    nv = y2.shape[0]
    wspec = pl.BlockSpec((128, 128), lambda v, i: (0, 0))
    bspec = pl.BlockSpec((128,), lambda v, i: (0,))
    return pl.pallas_call(
        _k6_body,
        grid=(nv, NB),
        in_specs=[
            pl.BlockSpec((1, 2, BN, 128), lambda v, i: (v, 0, i, 0)),
            pl.BlockSpec((1, BN, 128), lambda v, i: (v, i, 0)),
            pl.BlockSpec((1, BN, 1), lambda v, i: (v, i, 0)),
            bspec, wspec, bspec, wspec, bspec, wspec, bspec, wspec, bspec,
        ],
        out_specs=[
            pl.BlockSpec((1, BN, 128), lambda v, i: (v, i, 0)),
            pl.BlockSpec((1, BN, 128), lambda v, i: (v, i, 0)),
        ],
        out_shape=[
            jax.ShapeDtypeStruct((nv, N, 128), jnp.float32),
            jax.ShapeDtypeStruct((nv, N, 128), jnp.float32),
        ],
    )(seg2, y2, dinv, b1, Wi1, bi1, Wi2, bi2, Wc1, bc1, Wc2p, bc2p)


def _prep_edges(edge_index):
    pad = NW * EPW - E
    pad_idx = jnp.arange(pad, dtype=jnp.int32)
    srcp = jnp.concatenate(
        [edge_index[0], pad_idx % N]).reshape(NW, NCH, CH)
    dstp = jnp.concatenate(
        [edge_index[1], N + pad_idx % (NP - N)]).reshape(NW, NCH, CH)
    return srcp, dstp


def kernel(x1, edge_index1, x2, edge_index2, W0, b0, W1, b1,
           Wi1, bi1, Wi2, bi2, Wc1, bc1, Wc2, bc2):
    src1, dst1 = _prep_edges(edge_index1)
    src2, dst2 = _prep_edges(edge_index2)
    zeros128 = jnp.zeros((ZR, 128), jnp.float32)

    degpart = _deg_kernel(dst1, dst2).reshape(2, NC, NP, 1)
    dinv = _tc_dinv(degpart)

    x = jnp.stack([x1, x2])
    y1 = _tc_scale_matmul(x, dinv, W0)
    b0r = b0.reshape(2, 128)
    Wc2p = jnp.pad(Wc2, ((0, 0), (0, 128 - NCL)))
    bc2p = jnp.pad(bc2, (0, 128 - NCL))

    outs = []
    for v, (sv, dv) in enumerate(((src1, dst1), (src2, dst2))):
        s1 = _segL1(y1[v, 0], y1[v, 1], sv, sv, dv, dv, zeros128)
        y2 = _tc_layer2_in(s1, y1[v:v + 1], dinv[v:v + 1], W1, b0r)
        s2 = _segL2(y2[0], sv, sv, dv, dv, zeros128)
        zi, cp = _tc_heads(s2, y2, dinv[v:v + 1], b1, Wi1, bi1, Wi2, bi2,
                           Wc1, bc1, Wc2p, bc2p)
        outs.append((zi[0], cp[0, :, :NCL]))

    return (outs[0][0], outs[1][0], outs[0][1], outs[1][1])


# speedup vs baseline: 1.2638x; 1.0266x over previous
import functools

import jax
import jax.numpy as jnp
from jax import lax
from jax.experimental import pallas as pl
from jax.experimental.pallas import tpu as pltpu
from jax.experimental.pallas import tpu_sc as plsc

N = 10000
E = 320000
NCL = 3

NC = 2
NS = 16
NW = NC * NS
CH = 128
NCH = 80
EPW = NCH * CH
HCH = NCH // 2
NP = 10240
RPT = NP // NS
ZR = 640

_mesh = plsc.VectorSubcoreMesh(core_axis_name="c", subcore_axis_name="s")


def _sc_worker_ids():
    cid = lax.axis_index("c")
    sid = lax.axis_index("s")
    wid = sid * NC + cid
    return cid, sid, wid


@functools.partial(
    pl.kernel,
    mesh=_mesh,
    out_type=jax.ShapeDtypeStruct((2 * NC * NP,), jnp.float32),
    scratch_types=[
        pltpu.VMEM_SHARED((NP,), jnp.float32),
        pltpu.VMEM((NCH, CH), jnp.int32),
        pltpu.VMEM((CH,), jnp.float32),
        pltpu.VMEM((RPT,), jnp.float32),
    ],
)
def _deg_kernel(dst1, dst2, out, acc, dall, ones_v, zeros_v):
    cid, sid, wid = _sc_worker_ids()
    rbase = sid * RPT
    for j in range(CH // 16):
        ones_v[pl.ds(j * 16, 16)] = jnp.full((16,), 1.0, jnp.float32)
    for j in range(RPT // 16):
        zeros_v[pl.ds(j * 16, 16)] = jnp.zeros((16,), jnp.float32)
    for v, dref in ((0, dst1), (1, dst2)):
        pltpu.sync_copy(zeros_v, acc.at[pl.ds(rbase, RPT)])
        pltpu.sync_copy(dref.at[wid], dall)
        plsc.subcore_barrier()

        def chunk(i, carry):
            pltpu.sync_copy(ones_v, acc.at[dall.at[i]], add=True)
            return carry

        lax.fori_loop(0, NCH, chunk, 0)
        plsc.subcore_barrier()
        pltpu.sync_copy(acc.at[pl.ds(rbase, RPT)],
                        out.at[pl.ds((v * NC + cid) * NP + rbase, RPT)])
        plsc.subcore_barrier()


def _make_seg_kernel(n_pass, view_of_pass):
    def body(*refs):
        ys = refs[:n_pass]
        srcs = refs[n_pass:n_pass + 2]
        dsts = refs[n_pass + 2:n_pass + 4]
        zeros_h = refs[n_pass + 4]
        out = refs[n_pass + 5]
        acc, sall, dall, rows0, rows1, sem0, sem1 = refs[n_pass + 6:]
        cid, sid, wid = _sc_worker_ids()
        rbase = sid * RPT
        pltpu.sync_copy(zeros_h, acc.at[pl.ds(rbase, RPT)])
        plsc.subcore_barrier()
        for p in range(n_pass):
            v = view_of_pass[p]
            yref = ys[p]

            for half in range(2):
                cbase = half * HCH
                pltpu.sync_copy(srcs[v].at[wid, pl.ds(cbase, HCH)], sall)
                pltpu.sync_copy(dsts[v].at[wid, pl.ds(cbase, HCH)], dall)

                pltpu.async_copy(yref.at[sall.at[0]], rows0, sem0)

                def pair(t, carry, yref=yref):
                    c0 = 2 * t
                    pltpu.async_copy(yref.at[sall.at[c0 + 1]], rows1, sem1)
                    pltpu.make_async_copy(yref.at[sall.at[c0]], rows0,
                                          sem0).wait()
                    pltpu.sync_copy(rows0, acc.at[dall.at[c0]], add=True)
                    pltpu.async_copy(yref.at[sall.at[c0 + 2]], rows0, sem0)
                    pltpu.make_async_copy(yref.at[sall.at[c0 + 1]], rows1,
                                          sem1).wait()
                    pltpu.sync_copy(rows1, acc.at[dall.at[c0 + 1]], add=True)
                    return carry

                lax.fori_loop(0, HCH // 2 - 1, pair, 0)
                cl = HCH - 2
                pltpu.async_copy(yref.at[sall.at[cl + 1]], rows1, sem1)
                pltpu.make_async_copy(yref.at[sall.at[cl]], rows0, sem0).wait()
                pltpu.sync_copy(rows0, acc.at[dall.at[cl]], add=True)
                pltpu.make_async_copy(yref.at[sall.at[cl + 1]], rows1,
                                      sem1).wait()
                pltpu.sync_copy(rows1, acc.at[dall.at[cl + 1]], add=True)

            plsc.subcore_barrier()
            pltpu.sync_copy(acc.at[pl.ds(rbase, RPT)],
                            out.at[p, cid, pl.ds(rbase, RPT)])
            if p < n_pass - 1:
                pltpu.sync_copy(zeros_h, acc.at[pl.ds(rbase, RPT)])
            plsc.subcore_barrier()

    return pl.kernel(
        body,
        mesh=_mesh,
        out_type=jax.ShapeDtypeStruct((n_pass, NC, NP, 128), jnp.float32),
        scratch_types=[
            pltpu.VMEM_SHARED((NP, 128), jnp.float32),
            pltpu.VMEM((HCH, CH), jnp.int32),
            pltpu.VMEM((HCH, CH), jnp.int32),
            pltpu.VMEM((CH, 128), jnp.float32),
            pltpu.VMEM((CH, 128), jnp.float32),
            pltpu.SemaphoreType.DMA,
            pltpu.SemaphoreType.DMA,
        ],
    )


_segL1 = _make_seg_kernel(2, (0, 0))
_segL2 = _make_seg_kernel(1, (0,))

BN = 2000
NB = N // BN


BNP = 1024


def _dinv_body(dp_ref, dinv_ref):
    deg = dp_ref[0, 0] + dp_ref[0, 1] + 1.0
    dinv_ref[0] = lax.rsqrt(deg)


def _tc_dinv(degpart):
    return pl.pallas_call(
        _dinv_body,
        grid=(2, NP // BNP),
        in_specs=[pl.BlockSpec((1, 2, BNP, 1), lambda v, i: (v, 0, i, 0))],
        out_specs=pl.BlockSpec((1, BNP, 1), lambda v, i: (v, i, 0)),
        out_shape=jax.ShapeDtypeStruct((2, NP, 1), jnp.float32),
    )(degpart)


def _k2_body(x_ref, dv_ref, w0_ref, y_ref):
    dinv = dv_ref[0]
    xw = jnp.dot(x_ref[0], w0_ref[...], preferred_element_type=jnp.float32)
    y = xw * dinv
    y_ref[0, 0] = y[:, :128]
    y_ref[0, 1] = y[:, 128:]


def _k4_body(seg_ref, y_ref, dv_ref, w1_ref, b0_ref, y2_ref):
    dinv = dv_ref[0]
    hs = []
    for h in range(2):
        s = seg_ref[h, 0] + seg_ref[h, 1] + y_ref[0, h]
        hs.append(jnp.maximum(dinv * s + b0_ref[h], 0.0))
    hcat = jnp.concatenate(hs, axis=1)
    y2 = jnp.dot(hcat, w1_ref[...], preferred_element_type=jnp.float32)
    y2_ref[0] = y2 * dinv


def _k6_body(seg_ref, y2_ref, dv_ref, b1_ref, wi1_ref, bi1_ref, wi2_ref,
             bi2_ref, wc1_ref, bc1_ref, wc2_ref, bc2_ref, zi_ref, cp_ref):
    dinv = dv_ref[0]
    s = seg_ref[0, 0] + seg_ref[0, 1] + y2_ref[0]
    z = jnp.maximum(dinv * s + b1_ref[...], 0.0)
    hi = jnp.maximum(
        jnp.dot(z, wi1_ref[...], preferred_element_type=jnp.float32)
        + bi1_ref[...], 0.0)
    zi = (jnp.dot(hi, wi2_ref[...], preferred_element_type=jnp.float32)
          + bi2_ref[...])
    nrm = jnp.sqrt(jnp.sum(zi * zi, axis=1, keepdims=True))
    zi_ref[0] = zi / jnp.maximum(nrm, 1e-12)
    hc = jnp.maximum(
        jnp.dot(z, wc1_ref[...], preferred_element_type=jnp.float32)
        + bc1_ref[...], 0.0)
    lg = (jnp.dot(hc, wc2_ref[...], preferred_element_type=jnp.float32)
          + bc2_ref[...])
    col = lax.broadcasted_iota(jnp.int32, lg.shape, 1)
    lg = jnp.where(col < NCL, lg, -1e30)
    m = jnp.max(lg, axis=1, keepdims=True)
    e = jnp.exp(lg - m)
    cp_ref[0] = e / jnp.sum(e, axis=1, keepdims=True)


def _tc_scale_matmul(x, dinv, W0):
    nv = x.shape[0]
    return pl.pallas_call(
        _k2_body,
        grid=(nv, NB),
        in_specs=[
            pl.BlockSpec((1, BN, 128), lambda v, i: (v, i, 0)),
            pl.BlockSpec((1, BN, 1), lambda v, i: (v, i, 0)),
            pl.BlockSpec((128, 256), lambda v, i: (0, 0)),
        ],
        out_specs=pl.BlockSpec((1, 2, BN, 128), lambda v, i: (v, 0, i, 0)),
        out_shape=jax.ShapeDtypeStruct((nv, 2, N, 128), jnp.float32),
    )(x, dinv, W0)


def _tc_layer2_in(seg1, y1, dinv, W1, b0r):
    nv = y1.shape[0]
    return pl.pallas_call(
        _k4_body,
        grid=(nv, NB),
        in_specs=[
            pl.BlockSpec((2, 2, BN, 128), lambda v, i: (v, 0, i, 0)),
            pl.BlockSpec((1, 2, BN, 128), lambda v, i: (v, 0, i, 0)),
            pl.BlockSpec((1, BN, 1), lambda v, i: (v, i, 0)),
            pl.BlockSpec((256, 128), lambda v, i: (0, 0)),
            pl.BlockSpec((2, 128), lambda v, i: (0, 0)),
        ],
        out_specs=pl.BlockSpec((1, BN, 128), lambda v, i: (v, i, 0)),
        out_shape=jax.ShapeDtypeStruct((nv, N, 128), jnp.float32),
    )(seg1, y1, dinv, W1, b0r)


def _tc_heads(seg2, y2, dinv, b1, Wi1, bi1, Wi2, bi2, Wc1, bc1, Wc2p, bc2p):
    nv = y2.shape[0]
    wspec = pl.BlockSpec((128, 128), lambda v, i: (0, 0))
    bspec = pl.BlockSpec((128,), lambda v, i: (0,))
    return pl.pallas_call(
        _k6_body,
        grid=(nv, NB),
        in_specs=[
            pl.BlockSpec((1, 2, BN, 128), lambda v, i: (v, 0, i, 0)),
            pl.BlockSpec((1, BN, 128), lambda v, i: (v, i, 0)),
            pl.BlockSpec((1, BN, 1), lambda v, i: (v, i, 0)),
            bspec, wspec, bspec, wspec, bspec, wspec, bspec, wspec, bspec,
        ],
        out_specs=[
            pl.BlockSpec((1, BN, 128), lambda v, i: (v, i, 0)),
            pl.BlockSpec((1, BN, 128), lambda v, i: (v, i, 0)),
        ],
        out_shape=[
            jax.ShapeDtypeStruct((nv, N, 128), jnp.float32),
            jax.ShapeDtypeStruct((nv, N, 128), jnp.float32),
        ],
    )(seg2, y2, dinv, b1, Wi1, bi1, Wi2, bi2, Wc1, bc1, Wc2p, bc2p)


def _prep_edges(edge_index):
    pad = NW * EPW - E
    pad_idx = jnp.arange(pad, dtype=jnp.int32)
    srcp = jnp.concatenate(
        [edge_index[0], pad_idx % N]).reshape(NW, NCH, CH)
    dstp = jnp.concatenate(
        [edge_index[1], N + pad_idx % (NP - N)]).reshape(NW, NCH, CH)
    return srcp, dstp


def kernel(x1, edge_index1, x2, edge_index2, W0, b0, W1, b1,
           Wi1, bi1, Wi2, bi2, Wc1, bc1, Wc2, bc2):
    src1, dst1 = _prep_edges(edge_index1)
    src2, dst2 = _prep_edges(edge_index2)
    zeros128 = jnp.zeros((ZR, 128), jnp.float32)

    degpart = _deg_kernel(dst1, dst2).reshape(2, NC, NP, 1)
    dinv = _tc_dinv(degpart)

    b0r = b0.reshape(2, 128)
    Wc2p = jnp.pad(Wc2, ((0, 0), (0, 128 - NCL)))
    bc2p = jnp.pad(bc2, (0, 128 - NCL))

    outs = []
    for v, (xv, sv, dv) in enumerate(((x1, src1, dst1), (x2, src2, dst2))):
        y1 = _tc_scale_matmul(xv[None], dinv[v:v + 1], W0)
        s1 = _segL1(y1[0, 0], y1[0, 1], sv, sv, dv, dv, zeros128)
        y2 = _tc_layer2_in(s1, y1, dinv[v:v + 1], W1, b0r)
        s2 = _segL2(y2[0], sv, sv, dv, dv, zeros128)
        zi, cp = _tc_heads(s2, y2, dinv[v:v + 1], b1, Wi1, bi1, Wi2, bi2,
                           Wc1, bc1, Wc2p, bc2p)
        outs.append((zi[0], cp[0, :, :NCL]))

    return (outs[0][0], outs[1][0], outs[0][1], outs[1][1])
